# baseline (speedup 1.0000x reference)
import jax
import jax.numpy as jnp
from jax.experimental import pallas as pl
from jax.experimental.pallas import tpu as pltpu

_ROWS_PER_BLOCK = 400


def _gcn_body(nb, x_ref, adj_ref, w1_ref, b1_ref, w2_ref, b2_ref,
              out_ref, s_ref, g_ref):
    i = pl.program_id(0)
    r = adj_ref.shape[0]

    @pl.when(i == 0)
    def _():
        s_ref[:] = jnp.dot(x_ref[:], w1_ref[:],
                           preferred_element_type=jnp.float32)

    @pl.when(i < nb)
    def _():
        h = jnp.dot(adj_ref[:], s_ref[:],
                    preferred_element_type=jnp.float32) + b1_ref[:]
        h = jnp.maximum(h, 0.0)
        g_ref[pl.ds(i * r, r), :] = jnp.dot(
            h, w2_ref[:], preferred_element_type=jnp.float32)

    @pl.when(i >= nb)
    def _():
        z = jnp.dot(adj_ref[:], g_ref[:],
                    preferred_element_type=jnp.float32) + b2_ref[:]
        m = jnp.max(z, axis=1, keepdims=True)
        lse = jnp.log(jnp.sum(jnp.exp(z - m), axis=1, keepdims=True))
        out_ref[:] = z - m - lse


def kernel(x, adj, W1, b1, W2, b2):
    n, f = x.shape
    h_dim = W1.shape[1]
    c = W2.shape[1]
    r = _ROWS_PER_BLOCK
    nb = n // r

    def run(body):
        return pl.pallas_call(
            body,
            grid=(2 * nb,),
            in_specs=[
                pl.BlockSpec((n, f), lambda i: (0, 0)),
                pl.BlockSpec((r, n), lambda i: (jnp.where(i < nb, i, i - nb), 0)),
                pl.BlockSpec((f, h_dim), lambda i: (0, 0)),
                pl.BlockSpec((1, h_dim), lambda i: (0, 0)),
                pl.BlockSpec((h_dim, c), lambda i: (0, 0)),
                pl.BlockSpec((1, c), lambda i: (0, 0)),
            ],
            out_specs=pl.BlockSpec((r, c), lambda i: (jnp.maximum(i - nb, 0), 0)),
            out_shape=jax.ShapeDtypeStruct((n, c), jnp.float32),
            scratch_shapes=[
                pltpu.VMEM((n, h_dim), jnp.float32),
                pltpu.VMEM((n, c), jnp.float32),
            ],
        )(x, adj, W1, b1.reshape(1, h_dim), W2, b2.reshape(1, c))

    return run(lambda *refs: _gcn_body(nb, *refs))
